# baseline (speedup 1.0000x reference)
import jax
import jax.numpy as jnp
from jax.experimental import pallas as pl

B = 65536
BB = 128


def _body(x_ref, p_ref, o_ref):
    x = x_ref[...][:, None, :, :]
    t0 = p_ref[...][:, :, None, None]
    ci = jax.lax.broadcasted_iota(jnp.int32, (1, 3, 1, 1), 1)
    tgt = jnp.where(ci == 0, t0, jnp.where(ci == 1, 1 - t0, jnp.full_like(t0, 2)))
    o_ref[...] = (x == tgt).astype(jnp.float32)


def kernel(x, pls):
    pf = pls.reshape(B, 1)
    return pl.pallas_call(
        _body,
        grid=(B // BB,),
        in_specs=[
            pl.BlockSpec((BB, 19, 19), lambda i: (i, 0, 0)),
            pl.BlockSpec((BB, 1), lambda i: (i, 0)),
        ],
        out_specs=pl.BlockSpec((BB, 3, 19, 19), lambda i: (i, 0, 0, 0)),
        out_shape=jax.ShapeDtypeStruct((B, 3, 19, 19), jnp.float32),
    )(x, pf)


# speedup vs baseline: 2.5829x; 2.5829x over previous
import jax
import jax.numpy as jnp
from jax.experimental import pallas as pl

B = 65536
HW = 361
BB = 512


def _body(x_ref, p_ref, o_ref):
    x = x_ref[...][:, None, :]
    t0 = p_ref[...][:, :, None]
    ci = jax.lax.broadcasted_iota(jnp.int32, (1, 3, 1), 1)
    tgt = jnp.where(ci == 0, t0, jnp.where(ci == 1, 1 - t0, jnp.full_like(t0, 2)))
    o_ref[...] = (x == tgt).astype(jnp.float32)


def kernel(x, pls):
    xf = x.reshape(B, HW)
    pf = pls.reshape(B, 1)
    out = pl.pallas_call(
        _body,
        grid=(B // BB,),
        in_specs=[
            pl.BlockSpec((BB, HW), lambda i: (i, 0)),
            pl.BlockSpec((BB, 1), lambda i: (i, 0)),
        ],
        out_specs=pl.BlockSpec((BB, 3, HW), lambda i: (i, 0, 0)),
        out_shape=jax.ShapeDtypeStruct((B, 3, HW), jnp.float32),
    )(xf, pf)
    return out.reshape(B, 3, 19, 19)


# speedup vs baseline: 2.6769x; 1.0364x over previous
import jax
import jax.numpy as jnp
from jax.experimental import pallas as pl

B = 65536
HW = 361
BB = 1024


def _body(x_ref, p_ref, o_ref):
    x = x_ref[...][:, None, :]
    t0 = p_ref[...][:, :, None]
    ci = jax.lax.broadcasted_iota(jnp.int32, (1, 3, 1), 1)
    tgt = jnp.where(ci == 0, t0, jnp.where(ci == 1, 1 - t0, jnp.full_like(t0, 2)))
    o_ref[...] = (x == tgt).astype(jnp.float32)


def kernel(x, pls):
    xf = x.reshape(B, HW)
    pf = pls.reshape(B, 1)
    out = pl.pallas_call(
        _body,
        grid=(B // BB,),
        in_specs=[
            pl.BlockSpec((BB, HW), lambda i: (i, 0)),
            pl.BlockSpec((BB, 1), lambda i: (i, 0)),
        ],
        out_specs=pl.BlockSpec((BB, 3, HW), lambda i: (i, 0, 0)),
        out_shape=jax.ShapeDtypeStruct((B, 3, HW), jnp.float32),
    )(xf, pf)
    return out.reshape(B, 3, 19, 19)


# speedup vs baseline: 2.6928x; 1.0060x over previous
import jax
import jax.numpy as jnp
from jax.experimental import pallas as pl

B = 65536
HW = 361
BB = 2048


def _body(x_ref, p_ref, o_ref):
    x = x_ref[...][:, None, :]
    t0 = p_ref[...][:, :, None]
    ci = jax.lax.broadcasted_iota(jnp.int32, (1, 3, 1), 1)
    tgt = jnp.where(ci == 0, t0, jnp.where(ci == 1, 1 - t0, jnp.full_like(t0, 2)))
    o_ref[...] = (x == tgt).astype(jnp.float32)


def kernel(x, pls):
    xf = x.reshape(B, HW)
    pf = pls.reshape(B, 1)
    out = pl.pallas_call(
        _body,
        grid=(B // BB,),
        in_specs=[
            pl.BlockSpec((BB, HW), lambda i: (i, 0)),
            pl.BlockSpec((BB, 1), lambda i: (i, 0)),
        ],
        out_specs=pl.BlockSpec((BB, 3, HW), lambda i: (i, 0, 0)),
        out_shape=jax.ShapeDtypeStruct((B, 3, HW), jnp.float32),
    )(xf, pf)
    return out.reshape(B, 3, 19, 19)
